# baseline (speedup 1.0000x reference)
import jax
import jax.numpy as jnp
from jax import lax
from jax.experimental import pallas as pl
from jax.experimental.pallas import tpu as pltpu


def _relu_matmul_bias_t(x, wt_ref, b_ref):
    y = (
        lax.dot_general(
            jnp.maximum(x, 0.0),
            wt_ref[...],
            (((1,), (1,)), ((), ())),
            preferred_element_type=jnp.float32,
        )
        + b_ref[...]
    )
    return y.T


def _single_block_body(x_ref, wt_ref, b_ref, o_ref):
    o_ref[...] = _relu_matmul_bias_t(x_ref[...], wt_ref, b_ref)


def _make_pipelined_body(sizes, base):
    offs = [sum(sizes[:k]) for k in range(len(sizes))]
    num = len(sizes)

    has_tail = sizes[-1] % 128 != 0

    def body(
        x_hbm,
        wt_hbm,
        b_hbm,
        o_hbm,
        x_buf,
        y_buf,
        x_tail,
        y_tail,
        wt_ref,
        b_ref,
        in_sems,
        out_sems,
        wb_sems,
    ):
        def src_buf(k):
            if has_tail and k == num - 1:
                return x_tail
            return x_buf.at[k, pl.ds(0, sizes[k]), :]

        def dst_buf(k):
            if has_tail and k == num - 1:
                return y_tail
            return y_buf.at[k, :, pl.ds(0, sizes[k])]

        def get_in(k):
            return pltpu.make_async_copy(
                x_hbm.at[pl.ds(offs[k], sizes[k]), :],
                src_buf(k),
                in_sems.at[k],
            )

        def put_out(k):
            return pltpu.make_async_copy(
                dst_buf(k),
                o_hbm.at[:, pl.ds(offs[k], sizes[k])],
                out_sems.at[k],
            )

        get_w = pltpu.make_async_copy(wt_hbm, wt_ref, wb_sems.at[0])
        get_b = pltpu.make_async_copy(b_hbm, b_ref, wb_sems.at[1])
        get_in(0).start()
        get_w.start()
        get_b.start()
        for k in range(1, num):
            get_in(k).start()
        get_w.wait()
        get_b.wait()
        for k in range(num):
            get_in(k).wait()
            dst_buf(k)[...] = _relu_matmul_bias_t(src_buf(k)[...], wt_ref, b_ref)
            put_out(k).start()
        for k in range(num):
            put_out(k).wait()

    return body


def kernel(
    x_subject,
    x_region,
    edge_index_sr,
    edge_index_rr,
    edge_attr_sr,
    edge_attr_rr,
    sage_Wl0,
    sage_bl0,
    sage_Wr0,
    gcn_W0,
    gcn_b0,
    sage_Wl1,
    sage_bl1,
    sage_Wr1,
    gcn_W1,
    gcn_b1,
    lin_W,
    lin_b,
):
    m, d = x_subject.shape
    n = lin_W.shape[1]
    w_t = lin_W.T
    bias = lin_b.reshape(1, n)

    if m == 10000:
        sizes = [384, 768, 1536, 3072, 4224, 16]
    else:
        full = (m // 128) * 128
        sizes = [full // 2 // 128 * 128, full - full // 2 // 128 * 128]
        if m - full:
            sizes.append(m - full)
        sizes = [c for c in sizes if c]
    if all(c % 8 == 0 for c in sizes) and len(sizes) >= 2:
        num = len(sizes)
        has_tail = sizes[-1] % 128 != 0
        nfull = num - 1 if has_tail else num
        base = max(sizes[:nfull])
        tail = sizes[-1] if has_tail else 0
        out_t = pl.pallas_call(
            _make_pipelined_body(sizes, base),
            in_specs=[
                pl.BlockSpec(memory_space=pl.ANY),
                pl.BlockSpec(memory_space=pl.ANY),
                pl.BlockSpec(memory_space=pl.ANY),
            ],
            out_specs=pl.BlockSpec(memory_space=pl.ANY),
            out_shape=jax.ShapeDtypeStruct((n, m), jnp.float32),
            scratch_shapes=[
                pltpu.VMEM((nfull, base, d), jnp.float32),
                pltpu.VMEM((nfull, n, base), jnp.float32),
                pltpu.VMEM((max(tail, 8), d), jnp.float32),
                pltpu.VMEM((n, max(tail, 8)), jnp.float32),
                pltpu.VMEM((n, d), jnp.float32),
                pltpu.VMEM((1, n), jnp.float32),
                pltpu.SemaphoreType.DMA((num,)),
                pltpu.SemaphoreType.DMA((num,)),
                pltpu.SemaphoreType.DMA((2,)),
            ],
        )(x_subject, w_t, bias)
        return out_t.T

    out_t = pl.pallas_call(
        _single_block_body,
        grid=(1,),
        in_specs=[
            pl.BlockSpec((m, d), lambda i: (0, 0)),
            pl.BlockSpec((n, d), lambda i: (0, 0)),
            pl.BlockSpec((1, n), lambda i: (0, 0)),
        ],
        out_specs=pl.BlockSpec((n, m), lambda i: (0, 0)),
        out_shape=jax.ShapeDtypeStruct((n, m), jnp.float32),
        compiler_params=pltpu.CompilerParams(
            dimension_semantics=("arbitrary",),
        ),
    )(x_subject, w_t, bias)
    return out_t.T


# speedup vs baseline: 1.0515x; 1.0515x over previous
import jax
import jax.numpy as jnp
from jax import lax
from jax.experimental import pallas as pl
from jax.experimental.pallas import tpu as pltpu


def _relu_matmul_bias_t(x, wt_ref, b_ref):
    y = (
        lax.dot_general(
            jnp.maximum(x, 0.0),
            wt_ref[...],
            (((1,), (1,)), ((), ())),
            preferred_element_type=jnp.float32,
        )
        + b_ref[...]
    )
    return y.T


def _single_block_body(x_ref, wt_ref, b_ref, o_ref):
    o_ref[...] = _relu_matmul_bias_t(x_ref[...], wt_ref, b_ref)


def _make_pipelined_body(sizes, base):
    offs = [sum(sizes[:k]) for k in range(len(sizes))]
    num = len(sizes)

    has_tail = sizes[-1] % 128 != 0

    def body(
        x_hbm, wt_ref, b_ref, o_hbm, x_buf, y_buf, x_tail, y_tail, in_sems, out_sems
    ):
        def src_buf(k):
            if has_tail and k == num - 1:
                return x_tail
            return x_buf.at[k, pl.ds(0, sizes[k]), :]

        def dst_buf(k):
            if has_tail and k == num - 1:
                return y_tail
            return y_buf.at[k, :, pl.ds(0, sizes[k])]

        def get_in(k):
            return pltpu.make_async_copy(
                x_hbm.at[pl.ds(offs[k], sizes[k]), :],
                src_buf(k),
                in_sems.at[k],
            )

        def put_out(k):
            return pltpu.make_async_copy(
                dst_buf(k),
                o_hbm.at[:, pl.ds(offs[k], sizes[k])],
                out_sems.at[k],
            )

        for k in range(num):
            get_in(k).start()
        for k in range(num):
            get_in(k).wait()
            dst_buf(k)[...] = _relu_matmul_bias_t(src_buf(k)[...], wt_ref, b_ref)
            put_out(k).start()
        for k in range(num):
            put_out(k).wait()

    return body


def kernel(
    x_subject,
    x_region,
    edge_index_sr,
    edge_index_rr,
    edge_attr_sr,
    edge_attr_rr,
    sage_Wl0,
    sage_bl0,
    sage_Wr0,
    gcn_W0,
    gcn_b0,
    sage_Wl1,
    sage_bl1,
    sage_Wr1,
    gcn_W1,
    gcn_b1,
    lin_W,
    lin_b,
):
    m, d = x_subject.shape
    n = lin_W.shape[1]
    w_t = lin_W.T
    bias = lin_b.reshape(1, n)

    if m == 10000:
        sizes = [640, 1280, 3968, 4096, 16]
    else:
        full = (m // 128) * 128
        sizes = [full // 2 // 128 * 128, full - full // 2 // 128 * 128]
        if m - full:
            sizes.append(m - full)
        sizes = [c for c in sizes if c]
    if all(c % 8 == 0 for c in sizes) and len(sizes) >= 2:
        num = len(sizes)
        has_tail = sizes[-1] % 128 != 0
        nfull = num - 1 if has_tail else num
        base = max(sizes[:nfull])
        tail = sizes[-1] if has_tail else 0
        out_t = pl.pallas_call(
            _make_pipelined_body(sizes, base),
            in_specs=[
                pl.BlockSpec(memory_space=pl.ANY),
                pl.BlockSpec(memory_space=pltpu.VMEM),
                pl.BlockSpec(memory_space=pltpu.VMEM),
            ],
            out_specs=pl.BlockSpec(memory_space=pl.ANY),
            out_shape=jax.ShapeDtypeStruct((n, m), jnp.float32),
            scratch_shapes=[
                pltpu.VMEM((nfull, base, d), jnp.float32),
                pltpu.VMEM((nfull, n, base), jnp.float32),
                pltpu.VMEM((max(tail, 8), d), jnp.float32),
                pltpu.VMEM((n, max(tail, 8)), jnp.float32),
                pltpu.SemaphoreType.DMA((num,)),
                pltpu.SemaphoreType.DMA((num,)),
            ],
        )(x_subject, w_t, bias)
        return out_t.T

    out_t = pl.pallas_call(
        _single_block_body,
        grid=(1,),
        in_specs=[
            pl.BlockSpec((m, d), lambda i: (0, 0)),
            pl.BlockSpec((n, d), lambda i: (0, 0)),
            pl.BlockSpec((1, n), lambda i: (0, 0)),
        ],
        out_specs=pl.BlockSpec((n, m), lambda i: (0, 0)),
        out_shape=jax.ShapeDtypeStruct((n, m), jnp.float32),
        compiler_params=pltpu.CompilerParams(
            dimension_semantics=("arbitrary",),
        ),
    )(x_subject, w_t, bias)
    return out_t.T
